# baseline (speedup 1.0000x reference)
import functools

import jax
import jax.numpy as jnp
from jax import lax
from jax.experimental import pallas as pl
from jax.experimental.pallas import tpu as pltpu
from jax.experimental.pallas import tpu_sc as plsc

VOCAB = 100000
EMBED_DIM = 1024
SEQ_LEN = 8192

NC = 2
NS = 16
NW = NC * NS
B_PER_W = SEQ_LEN // NW
CHUNK = 32
NCH = B_PER_W // CHUNK


NBUF = 3


def _emb_body(ids_hbm, table_hbm, out_hbm, idx_v, rows, gsems, osems):
    wid = lax.axis_index("s") * NC + lax.axis_index("c")
    base = wid * B_PER_W
    pltpu.sync_copy(ids_hbm.at[wid], idx_v)
    gd = [None] * NCH
    od = [None] * NCH
    for c in range(min(NBUF, NCH)):
        gd[c] = pltpu.async_copy(table_hbm.at[idx_v.at[c]], rows.at[c % NBUF],
                                 gsems[c % NBUF])
    for c in range(NCH):
        b = c % NBUF
        if c >= 1 and (c - 1) + NBUF < NCH:
            od[c - 1].wait()
            pb = (c - 1) % NBUF
            gd[c - 1 + NBUF] = pltpu.async_copy(
                table_hbm.at[idx_v.at[c - 1 + NBUF]], rows.at[pb], gsems[pb])
        gd[c].wait()
        od[c] = pltpu.async_copy(rows.at[b],
                                 out_hbm.at[pl.ds(base + c * CHUNK, CHUNK)],
                                 osems[b])
    for c in range(max(0, NCH - NBUF), NCH):
        od[c].wait()


@jax.jit
def kernel(input_ids, embed_tokens_weight):
    ids = input_ids.reshape(NW, NCH, CHUNK)
    call = pl.kernel(
        _emb_body,
        out_type=jax.ShapeDtypeStruct((SEQ_LEN, EMBED_DIM), jnp.float32),
        mesh=plsc.VectorSubcoreMesh(core_axis_name="c", subcore_axis_name="s"),
        scratch_types=[
            pltpu.VMEM((NCH, CHUNK), jnp.int32),
            pltpu.VMEM((NBUF, CHUNK, EMBED_DIM), jnp.float32),
            [pltpu.SemaphoreType.DMA] * NBUF,
            [pltpu.SemaphoreType.DMA] * NBUF,
        ],
    )
    out = call(ids, embed_tokens_weight)
    return out.reshape(1, SEQ_LEN, EMBED_DIM)


# speedup vs baseline: 1.0045x; 1.0045x over previous
import functools

import jax
import jax.numpy as jnp
from jax import lax
from jax.experimental import pallas as pl
from jax.experimental.pallas import tpu as pltpu
from jax.experimental.pallas import tpu_sc as plsc

VOCAB = 100000
EMBED_DIM = 1024
SEQ_LEN = 8192

NC = 2
NS = 16
NW = NC * NS
B_PER_W = SEQ_LEN // NW
CHUNK = 16
NCH = B_PER_W // CHUNK


NBUF = 6


def _emb_body(ids_hbm, table_hbm, out_hbm, idx_v, rows, gsems, osems):
    wid = lax.axis_index("s") * NC + lax.axis_index("c")
    base = wid * B_PER_W
    pltpu.sync_copy(ids_hbm.at[wid], idx_v)
    gd = [None] * NCH
    od = [None] * NCH
    for c in range(min(NBUF, NCH)):
        gd[c] = pltpu.async_copy(table_hbm.at[idx_v.at[c]], rows.at[c % NBUF],
                                 gsems[c % NBUF])
    for c in range(NCH):
        b = c % NBUF
        if c >= 1 and (c - 1) + NBUF < NCH:
            od[c - 1].wait()
            pb = (c - 1) % NBUF
            gd[c - 1 + NBUF] = pltpu.async_copy(
                table_hbm.at[idx_v.at[c - 1 + NBUF]], rows.at[pb], gsems[pb])
        gd[c].wait()
        od[c] = pltpu.async_copy(rows.at[b],
                                 out_hbm.at[pl.ds(base + c * CHUNK, CHUNK)],
                                 osems[b])
    for c in range(max(0, NCH - NBUF), NCH):
        od[c].wait()


@jax.jit
def kernel(input_ids, embed_tokens_weight):
    ids = input_ids.reshape(NW, NCH, CHUNK)
    call = pl.kernel(
        _emb_body,
        out_type=jax.ShapeDtypeStruct((SEQ_LEN, EMBED_DIM), jnp.float32),
        mesh=plsc.VectorSubcoreMesh(core_axis_name="c", subcore_axis_name="s"),
        scratch_types=[
            pltpu.VMEM((NCH, CHUNK), jnp.int32),
            pltpu.VMEM((NBUF, CHUNK, EMBED_DIM), jnp.float32),
            [pltpu.SemaphoreType.DMA] * NBUF,
            [pltpu.SemaphoreType.DMA] * NBUF,
        ],
    )
    out = call(ids, embed_tokens_weight)
    return out.reshape(1, SEQ_LEN, EMBED_DIM)
